# baseline (speedup 1.0000x reference)
import functools

import jax
import jax.numpy as jnp
from jax import lax
from jax.experimental import pallas as pl
from jax.experimental.pallas import tpu as pltpu
from jax.experimental.pallas import tpu_sc as plsc

N_NODES = 10000
N_HE = 10000
D_IN = 128
DH = 64
R = 10240
PAD_BIN = 10000
INC = 320000
INC_PAD = 327680
IDX_ROWS = 2560
NS = 16
RT = R // NS
IRT = IDX_ROWS // NS
WIN = 16
NWIN = IRT // WIN
EPS = 1e-6


def _mm_body(x_ref, w_ref, b_ref, o_ref):
    o_ref[0] = lax.dot_general(
        x_ref[...], w_ref[...], (((1,), (1,)), ((), ())),
        preferred_element_type=jnp.float32) + b_ref[0, 0][None, :]


def _transform(x_pad, W, b2):
    return pl.pallas_call(
        _mm_body,
        grid=(2, 4),
        in_specs=[
            pl.BlockSpec((R // 4, 128), lambda c, r: (r, 0)),
            pl.BlockSpec((DH, 128), lambda c, r: (c, 0)),
            pl.BlockSpec((1, 1, DH), lambda c, r: (c, 0, 0)),
        ],
        out_specs=pl.BlockSpec((1, R // 4, DH), lambda c, r: (c, r, 0)),
        out_shape=jax.ShapeDtypeStruct((2, R, DH), jnp.float32),
    )(x_pad, W, b2)


def _sc_body(xt_ref, idxn_ref, idxe_ref, w_ref,
             out_ref,
             tab_sh, acc_sh, b_sh, d_sh,
             gwin, swin, buf_all, ones2, chunk_v, svec, wvec,
             gsems, ssems, hsem):
    cid = lax.axis_index("c")
    sid = lax.axis_index("s")
    row0 = sid * RT
    ib0 = sid * IRT

    zeros16 = jnp.zeros((16,), jnp.float32)
    ones16 = jnp.ones((16,), jnp.float32)

    for k in range(8):
        ones2[0, pl.ds(16 * k, 16)] = ones16

    def zchunk(i, c):
        for k in range(DH // 16):
            chunk_v[i, pl.ds(16 * k, 16)] = zeros16
        return c

    def zero_acc(also_hist):
        lax.fori_loop(0, 64, zchunk, 0)

        def zacc(j, c):
            r = row0 + j * 64
            pltpu.async_copy(chunk_v, acc_sh.at[pl.ds(r, 64)], hsem)
            if also_hist:
                pltpu.async_copy(chunk_v.at[0], b_sh.at[pl.ds(r, 64)], hsem)
                pltpu.async_copy(chunk_v.at[0], d_sh.at[pl.ds(r, 64)], hsem)
            return c
        lax.fori_loop(0, RT // 64, zacc, 0)

        def zdrain(j, c):
            r = row0 + j * 64
            pltpu.make_async_copy(chunk_v, acc_sh.at[pl.ds(r, 64)],
                                  hsem).wait()
            if also_hist:
                pltpu.make_async_copy(chunk_v.at[0], b_sh.at[pl.ds(r, 64)],
                                      hsem).wait()
                pltpu.make_async_copy(chunk_v.at[0], d_sh.at[pl.ds(r, 64)],
                                      hsem).wait()
            return c
        lax.fori_loop(0, RT // 64, zdrain, 0)

    pltpu.sync_copy(xt_ref.at[cid].at[pl.ds(row0, RT)],
                    tab_sh.at[pl.ds(row0, RT)])
    zero_acc(True)
    plsc.subcore_barrier()

    def bv(L):
        return buf_all.at[pl.ds(128 * L, 128)]

    def run_phase(gidx_ref, sidx_ref, with_hist):
        def stage(s, c):
            wbase = ib0 + s * WIN
            pltpu.sync_copy(gidx_ref.at[pl.ds(wbase, WIN)], gwin)
            pltpu.sync_copy(sidx_ref.at[pl.ds(wbase, WIN)], swin)
            for L in range(2):
                pltpu.async_copy(tab_sh.at[gwin.at[L]], bv(L), gsems.at[L])
            for j in range(WIN):
                L = j % 4
                Lg = (j + 2) % 4
                if j >= 2:
                    pltpu.make_async_copy(
                        bv(Lg), acc_sh.at[swin.at[j - 2]],
                        ssems.at[Lg]).wait()
                if j + 2 < WIN:
                    pltpu.async_copy(tab_sh.at[gwin.at[j + 2]], bv(Lg),
                                     gsems.at[Lg])
                pltpu.make_async_copy(tab_sh.at[gwin.at[j]], bv(L),
                                      gsems.at[L]).wait()
                pltpu.async_copy(bv(L), acc_sh.at[swin.at[j]], ssems.at[L],
                                 add=True)
                if with_hist:
                    pltpu.async_copy(ones2.at[0], d_sh.at[gwin.at[j]],
                                     hsem, add=True)
                    pltpu.async_copy(ones2.at[0], b_sh.at[swin.at[j]],
                                     hsem, add=True)
            for j in (WIN - 2, WIN - 1):
                pltpu.make_async_copy(bv(j % 4), acc_sh.at[swin.at[j]],
                                      ssems.at[j % 4]).wait()
            if with_hist:
                for j in range(WIN):
                    pltpu.make_async_copy(ones2.at[0], d_sh.at[gwin.at[j]],
                                          hsem).wait()
                    pltpu.make_async_copy(ones2.at[0], b_sh.at[swin.at[j]],
                                          hsem).wait()
            return c
        lax.fori_loop(0, NWIN, stage, 0)

    run_phase(idxn_ref, idxe_ref, True)
    plsc.subcore_barrier()

    def scale_he(j, c):
        r = row0 + j * 64
        pltpu.sync_copy(acc_sh.at[pl.ds(r, 64)], chunk_v)
        pltpu.sync_copy(b_sh.at[pl.ds(r, 64)], svec)
        pltpu.sync_copy(w_ref.at[pl.ds(r, 64)], wvec)
        for k in range(4):
            sl = pl.ds(16 * k, 16)
            svec[sl] = wvec[sl] / (svec[sl] + EPS)

        def grpmul(g, c2):
            s16 = svec[pl.ds(16 * g, 16)]
            for rr in range(16):
                srow = jnp.broadcast_to(s16[rr], (16,))
                row = 16 * g + rr
                for k in range(DH // 16):
                    sl = pl.ds(16 * k, 16)
                    chunk_v[row, sl] = chunk_v[row, sl] * srow
            return c2
        lax.fori_loop(0, 4, grpmul, 0)
        pltpu.sync_copy(chunk_v, tab_sh.at[pl.ds(r, 64)])
        return c
    lax.fori_loop(0, RT // 64, scale_he, 0)
    plsc.subcore_barrier()

    zero_acc(False)
    plsc.subcore_barrier()

    run_phase(idxe_ref, idxn_ref, False)
    plsc.subcore_barrier()

    def scale_out(j, c):
        r = row0 + j * 64
        pltpu.sync_copy(acc_sh.at[pl.ds(r, 64)], chunk_v)
        pltpu.sync_copy(d_sh.at[pl.ds(r, 64)], svec)
        for k in range(4):
            sl = pl.ds(16 * k, 16)
            svec[sl] = 1.0 / (svec[sl] + EPS)

        def grpmul(g, c2):
            s16 = svec[pl.ds(16 * g, 16)]
            for rr in range(16):
                srow = jnp.broadcast_to(s16[rr], (16,))
                row = 16 * g + rr
                for k in range(DH // 16):
                    sl = pl.ds(16 * k, 16)
                    chunk_v[row, sl] = chunk_v[row, sl] * srow
            return c2
        lax.fori_loop(0, 4, grpmul, 0)
        pltpu.sync_copy(chunk_v,
                        out_ref.at[pl.ds(r, 64), pl.ds(cid * DH, DH)])
        return c
    lax.fori_loop(0, RT // 64, scale_out, 0)


_sc_call = functools.partial(
    pl.kernel,
    out_type=jax.ShapeDtypeStruct((R, D_IN), jnp.float32),
    mesh=plsc.VectorSubcoreMesh(core_axis_name="c", subcore_axis_name="s"),
    compiler_params=pltpu.CompilerParams(use_tc_tiling_on_sc=False),
    scratch_types=[
        pltpu.VMEM_SHARED((R, DH), jnp.float32),
        pltpu.VMEM_SHARED((R, DH), jnp.float32),
        pltpu.VMEM_SHARED((R,), jnp.float32),
        pltpu.VMEM_SHARED((R,), jnp.float32),
        pltpu.VMEM((WIN, 128), jnp.int32),
        pltpu.VMEM((WIN, 128), jnp.int32),
        pltpu.VMEM((4 * 128, DH), jnp.float32),
        pltpu.VMEM((1, 128), jnp.float32),
        pltpu.VMEM((64, DH), jnp.float32),
        pltpu.VMEM((64,), jnp.float32),
        pltpu.VMEM((64,), jnp.float32),
        pltpu.SemaphoreType.DMA((4,)),
        pltpu.SemaphoreType.DMA((4,)),
        pltpu.SemaphoreType.DMA,
    ],
)(_sc_body)


def kernel(x, hyperedge_index, W, b, hyperedge_weight):
    x_pad = jnp.pad(x, ((0, R - N_NODES), (0, 0)))
    pad = jnp.full((INC_PAD - INC,), PAD_BIN, jnp.int32)
    idxn = jnp.concatenate([hyperedge_index[0], pad]).reshape(IDX_ROWS, 128)
    idxe = jnp.concatenate([hyperedge_index[1], pad]).reshape(IDX_ROWS, 128)
    w_pad = jnp.pad(hyperedge_weight, (0, R - N_HE))
    b2 = b.reshape(2, 1, DH)
    xt = _transform(x_pad, W, b2)
    out2 = _sc_call(xt, idxn, idxe, w_pad)
    return out2[:N_NODES]


# speedup vs baseline: 1.0574x; 1.0574x over previous
import functools

import jax
import jax.numpy as jnp
from jax import lax
from jax.experimental import pallas as pl
from jax.experimental.pallas import tpu as pltpu
from jax.experimental.pallas import tpu_sc as plsc

N_NODES = 10000
N_HE = 10000
D_IN = 128
DH = 64
R = 10240
PAD_BIN = 10000
INC = 320000
INC_PAD = 327680
IDX_ROWS = 2560
NS = 16
RT = R // NS
IRT = IDX_ROWS // NS
WIN = 32
NWIN = IRT // WIN
EPS = 1e-6


def _mm_body(x_ref, w_ref, b_ref, o_ref):
    o_ref[0] = lax.dot_general(
        x_ref[...], w_ref[...], (((1,), (1,)), ((), ())),
        preferred_element_type=jnp.float32) + b_ref[0, 0][None, :]


def _transform(x_pad, W, b2):
    return pl.pallas_call(
        _mm_body,
        grid=(2, 4),
        in_specs=[
            pl.BlockSpec((R // 4, 128), lambda c, r: (r, 0)),
            pl.BlockSpec((DH, 128), lambda c, r: (c, 0)),
            pl.BlockSpec((1, 1, DH), lambda c, r: (c, 0, 0)),
        ],
        out_specs=pl.BlockSpec((1, R // 4, DH), lambda c, r: (c, r, 0)),
        out_shape=jax.ShapeDtypeStruct((2, R, DH), jnp.float32),
    )(x_pad, W, b2)


def _sc_body(xt_ref, idxn_ref, idxe_ref, w_ref,
             out_ref,
             tab_sh, acc_sh, b_sh, d_sh,
             gwin, swin, buf_all, ones2, chunk_v, svec, wvec,
             gsems, ssems, hsem):
    cid = lax.axis_index("c")
    sid = lax.axis_index("s")
    row0 = sid * RT
    ib0 = sid * IRT

    zeros16 = jnp.zeros((16,), jnp.float32)
    ones16 = jnp.ones((16,), jnp.float32)

    for k in range(8):
        ones2[0, pl.ds(16 * k, 16)] = ones16

    def zchunk(i, c):
        for k in range(DH // 16):
            chunk_v[i, pl.ds(16 * k, 16)] = zeros16
        return c

    def zero_acc(also_hist):
        lax.fori_loop(0, 64, zchunk, 0)

        def zacc(j, c):
            r = row0 + j * 64
            pltpu.async_copy(chunk_v, acc_sh.at[pl.ds(r, 64)], hsem)
            if also_hist:
                pltpu.async_copy(chunk_v.at[0], b_sh.at[pl.ds(r, 64)], hsem)
                pltpu.async_copy(chunk_v.at[0], d_sh.at[pl.ds(r, 64)], hsem)
            return c
        lax.fori_loop(0, RT // 64, zacc, 0)

        def zdrain(j, c):
            r = row0 + j * 64
            pltpu.make_async_copy(chunk_v, acc_sh.at[pl.ds(r, 64)],
                                  hsem).wait()
            if also_hist:
                pltpu.make_async_copy(chunk_v.at[0], b_sh.at[pl.ds(r, 64)],
                                      hsem).wait()
                pltpu.make_async_copy(chunk_v.at[0], d_sh.at[pl.ds(r, 64)],
                                      hsem).wait()
            return c
        lax.fori_loop(0, RT // 64, zdrain, 0)

    pltpu.sync_copy(xt_ref.at[cid].at[pl.ds(row0, RT)],
                    tab_sh.at[pl.ds(row0, RT)])
    zero_acc(True)
    plsc.subcore_barrier()

    def bv(L):
        return buf_all.at[pl.ds(128 * L, 128)]

    def run_phase(gidx_ref, sidx_ref, with_hist):
        def stage(s, c):
            wbase = ib0 + s * WIN
            pltpu.sync_copy(gidx_ref.at[pl.ds(wbase, WIN)], gwin)
            pltpu.sync_copy(sidx_ref.at[pl.ds(wbase, WIN)], swin)
            for L in range(2):
                pltpu.async_copy(tab_sh.at[gwin.at[L]], bv(L), gsems.at[L])
            for j in range(WIN):
                L = j % 4
                Lg = (j + 2) % 4
                if j >= 2:
                    pltpu.make_async_copy(
                        bv(Lg), acc_sh.at[swin.at[j - 2]],
                        ssems.at[Lg]).wait()
                if j + 2 < WIN:
                    pltpu.async_copy(tab_sh.at[gwin.at[j + 2]], bv(Lg),
                                     gsems.at[Lg])
                pltpu.make_async_copy(tab_sh.at[gwin.at[j]], bv(L),
                                      gsems.at[L]).wait()
                pltpu.async_copy(bv(L), acc_sh.at[swin.at[j]], ssems.at[L],
                                 add=True)
                if with_hist:
                    pltpu.async_copy(ones2.at[0], d_sh.at[gwin.at[j]],
                                     hsem, add=True)
                    pltpu.async_copy(ones2.at[0], b_sh.at[swin.at[j]],
                                     hsem, add=True)
            for j in (WIN - 2, WIN - 1):
                pltpu.make_async_copy(bv(j % 4), acc_sh.at[swin.at[j]],
                                      ssems.at[j % 4]).wait()
            if with_hist:
                for j in range(WIN):
                    pltpu.make_async_copy(ones2.at[0], d_sh.at[gwin.at[j]],
                                          hsem).wait()
                    pltpu.make_async_copy(ones2.at[0], b_sh.at[swin.at[j]],
                                          hsem).wait()
            return c
        lax.fori_loop(0, NWIN, stage, 0)

    run_phase(idxn_ref, idxe_ref, True)
    plsc.subcore_barrier()

    def scale_he(j, c):
        r = row0 + j * 64
        pltpu.sync_copy(acc_sh.at[pl.ds(r, 64)], chunk_v)
        pltpu.sync_copy(b_sh.at[pl.ds(r, 64)], svec)
        pltpu.sync_copy(w_ref.at[pl.ds(r, 64)], wvec)
        for k in range(4):
            sl = pl.ds(16 * k, 16)
            svec[sl] = wvec[sl] / (svec[sl] + EPS)

        def grpmul(g, c2):
            s16 = svec[pl.ds(16 * g, 16)]
            for rr in range(16):
                srow = jnp.broadcast_to(s16[rr], (16,))
                row = 16 * g + rr
                for k in range(DH // 16):
                    sl = pl.ds(16 * k, 16)
                    chunk_v[row, sl] = chunk_v[row, sl] * srow
            return c2
        lax.fori_loop(0, 4, grpmul, 0)
        pltpu.sync_copy(chunk_v, tab_sh.at[pl.ds(r, 64)])
        return c
    lax.fori_loop(0, RT // 64, scale_he, 0)
    plsc.subcore_barrier()

    zero_acc(False)
    plsc.subcore_barrier()

    run_phase(idxe_ref, idxn_ref, False)
    plsc.subcore_barrier()

    def scale_out(j, c):
        r = row0 + j * 64
        pltpu.sync_copy(acc_sh.at[pl.ds(r, 64)], chunk_v)
        pltpu.sync_copy(d_sh.at[pl.ds(r, 64)], svec)
        for k in range(4):
            sl = pl.ds(16 * k, 16)
            svec[sl] = 1.0 / (svec[sl] + EPS)

        def grpmul(g, c2):
            s16 = svec[pl.ds(16 * g, 16)]
            for rr in range(16):
                srow = jnp.broadcast_to(s16[rr], (16,))
                row = 16 * g + rr
                for k in range(DH // 16):
                    sl = pl.ds(16 * k, 16)
                    chunk_v[row, sl] = chunk_v[row, sl] * srow
            return c2
        lax.fori_loop(0, 4, grpmul, 0)
        pltpu.sync_copy(chunk_v,
                        out_ref.at[pl.ds(r, 64), pl.ds(cid * DH, DH)])
        return c
    lax.fori_loop(0, RT // 64, scale_out, 0)


_sc_call = functools.partial(
    pl.kernel,
    out_type=jax.ShapeDtypeStruct((R, D_IN), jnp.float32),
    mesh=plsc.VectorSubcoreMesh(core_axis_name="c", subcore_axis_name="s"),
    compiler_params=pltpu.CompilerParams(use_tc_tiling_on_sc=False),
    scratch_types=[
        pltpu.VMEM_SHARED((R, DH), jnp.float32),
        pltpu.VMEM_SHARED((R, DH), jnp.float32),
        pltpu.VMEM_SHARED((R,), jnp.float32),
        pltpu.VMEM_SHARED((R,), jnp.float32),
        pltpu.VMEM((WIN, 128), jnp.int32),
        pltpu.VMEM((WIN, 128), jnp.int32),
        pltpu.VMEM((4 * 128, DH), jnp.float32),
        pltpu.VMEM((1, 128), jnp.float32),
        pltpu.VMEM((64, DH), jnp.float32),
        pltpu.VMEM((64,), jnp.float32),
        pltpu.VMEM((64,), jnp.float32),
        pltpu.SemaphoreType.DMA((4,)),
        pltpu.SemaphoreType.DMA((4,)),
        pltpu.SemaphoreType.DMA,
    ],
)(_sc_body)


def kernel(x, hyperedge_index, W, b, hyperedge_weight):
    x_pad = jnp.pad(x, ((0, R - N_NODES), (0, 0)))
    pad = jnp.full((INC_PAD - INC,), PAD_BIN, jnp.int32)
    idxn = jnp.concatenate([hyperedge_index[0], pad]).reshape(IDX_ROWS, 128)
    idxe = jnp.concatenate([hyperedge_index[1], pad]).reshape(IDX_ROWS, 128)
    w_pad = jnp.pad(hyperedge_weight, (0, R - N_HE))
    b2 = b.reshape(2, 1, DH)
    xt = _transform(x_pad, W, b2)
    out2 = _sc_call(xt, idxn, idxe, w_pad)
    return out2[:N_NODES]
